# baseline (speedup 1.0000x reference)
import functools

import jax
import jax.numpy as jnp
from jax.experimental import pallas as pl
from jax.experimental.pallas import tpu as pltpu

_EPS = 1e-10
_BIG = 2147483647


def _gmb(u):
    return -jnp.log(-jnp.log(u + _EPS) + _EPS)


def _body(dl_ref, pol_ref, t_hr, t_sbp, t_glu, t_po,
          ud_ref, u_hr, u_sbp, u_glu, u_po, up_ref,
          samples_ref, actions_ref, logp_ref,
          diab_s, *vs, V, C, N, B):
    groups = [tuple(vs[5 * k + j] for j in range(5)) for k in range(4)]
    i = pl.program_id(0)

    @pl.when(i == 0)
    def _init():
        dl = dl_ref[...]
        zd = dl + _gmb(ud_ref[...])
        s0 = (zd[:, 1:2] > zd[:, 0:1]).astype(jnp.int32)
        diab_s[...] = s0
        m2 = jnp.max(dl)
        lse2 = m2 + jnp.log(jnp.sum(jnp.exp(dl - m2)))
        samples_ref[:, 0:1] = s0
        logp_ref[:, 0:1] = jnp.where(s0 == 1, dl[0, 1], dl[0, 0]) - lse2

        pv = pol_ref[...]
        zp = pv + _gmb(up_ref[...])
        a = jnp.argmax(zp, axis=1).astype(jnp.int32)[:, None]
        actions_ref[...] = a
        mp = jnp.max(pv)
        lsep = mp + jnp.log(jnp.sum(jnp.exp(pv - mp)))
        ia8 = jax.lax.broadcasted_iota(jnp.int32, zp.shape, 1)
        tvp = jnp.sum(jnp.where(ia8 == a, pv, 0.0), axis=1, keepdims=True)
        logp_ref[:, 5:6] = tvp - lsep

        for (rmax, ridx, rtv, rm, rs) in groups:
            rmax[...] = jnp.full(rmax.shape, -jnp.inf, jnp.float32)
            ridx[...] = jnp.zeros(ridx.shape, jnp.int32)
            rtv[...] = jnp.zeros(rtv.shape, jnp.float32)
            rm[...] = jnp.full(rm.shape, -jnp.inf, jnp.float32)
            rs[...] = jnp.zeros(rs.shape, jnp.float32)

    diab = diab_s[...]
    base = i * C
    col = base + jax.lax.broadcasted_iota(jnp.int32, (B, C), 1)
    valid = col < V
    col2 = jax.lax.broadcasted_iota(jnp.int32, (2, C), 1)
    valid2 = (base + col2) < V

    for (t_ref, u_ref), (rmax, ridx, rtv, rm, rs) in zip(
            [(t_hr, u_hr), (t_sbp, u_sbp), (t_glu, u_glu), (t_po, u_po)],
            groups):
        g = _gmb(u_ref[...])
        t = t_ref[...]
        tsel = jnp.where(diab == 1, t[1:2, :], t[0:1, :])
        z = jnp.where(valid, tsel + g, -jnp.inf)
        upd = z > rmax[...]
        rmax[...] = jnp.where(upd, z, rmax[...])
        ridx[...] = jnp.where(upd, col, ridx[...])
        rtv[...] = jnp.where(upd, tsel, rtv[...])
        tmsk = jnp.where(valid2, t, -jnp.inf)
        m_old = rm[...]
        m_new = jnp.maximum(m_old, tmsk)
        rs[...] = rs[...] * jnp.exp(m_old - m_new) + \
            jnp.where(valid2, jnp.exp(t - m_new), 0.0)
        rm[...] = m_new

    @pl.when(i == N - 1)
    def _fin():
        diab_f = diab_s[...]
        for v, (rmax, ridx, rtv, rm, rs) in enumerate(groups):
            mv = rmax[...]
            iv = ridx[...]
            maxv = jnp.max(mv, axis=1, keepdims=True)
            at_max = mv == maxv
            bj = jnp.min(jnp.where(at_max, iv, _BIG),
                         axis=1, keepdims=True)
            sel_lane = at_max & (iv == bj)
            tv = jnp.sum(jnp.where(sel_lane, rtv[...], 0.0),
                         axis=1, keepdims=True)
            samples_ref[:, v + 1:v + 2] = bj
            m_l = rm[...]
            m_g = jnp.max(m_l, axis=1, keepdims=True)
            s_g = jnp.sum(rs[...] * jnp.exp(m_l - m_g),
                          axis=1, keepdims=True)
            lse = m_g + jnp.log(s_g)
            lse_sel = jnp.where(diab_f == 1, lse[1, 0], lse[0, 0])
            logp_ref[:, v + 1:v + 2] = tv - lse_sel


def kernel(s0_diab_logits, s0_hr, s0_sysbp, s0_glucose, s0_percoxyg,
           policy_logits, u_diab, u_hr, u_sysbp, u_glucose, u_percoxyg,
           u_policy):
    B, V = u_hr.shape
    A = u_policy.shape[1]
    C = 512
    N = pl.cdiv(V, C)
    dl = s0_diab_logits.reshape(1, 2)
    pol = policy_logits.reshape(1, A)

    const2 = lambda i: (0, 0)
    tspec = pl.BlockSpec((2, C), lambda i: (0, i))
    uspec = pl.BlockSpec((B, C), lambda i: (0, i))

    scratch = [pltpu.VMEM((B, 1), jnp.int32)]
    for _ in range(4):
        scratch += [pltpu.VMEM((B, C), jnp.float32),
                    pltpu.VMEM((B, C), jnp.int32),
                    pltpu.VMEM((B, C), jnp.float32),
                    pltpu.VMEM((2, C), jnp.float32),
                    pltpu.VMEM((2, C), jnp.float32)]

    samples, actions2, logp = pl.pallas_call(
        functools.partial(_body, V=V, C=C, N=N, B=B),
        grid=(N,),
        in_specs=[
            pl.BlockSpec((1, 2), const2),
            pl.BlockSpec((1, A), const2),
            tspec, tspec, tspec, tspec,
            pl.BlockSpec((B, 2), const2),
            uspec, uspec, uspec, uspec,
            pl.BlockSpec((B, A), const2),
        ],
        out_specs=[
            pl.BlockSpec((B, 5), const2),
            pl.BlockSpec((B, 1), const2),
            pl.BlockSpec((B, 6), const2),
        ],
        out_shape=[
            jax.ShapeDtypeStruct((B, 5), jnp.int32),
            jax.ShapeDtypeStruct((B, 1), jnp.int32),
            jax.ShapeDtypeStruct((B, 6), jnp.float32),
        ],
        scratch_shapes=scratch,
        compiler_params=pltpu.CompilerParams(
            dimension_semantics=("arbitrary",)),
    )(dl, pol, s0_hr, s0_sysbp, s0_glucose, s0_percoxyg,
      u_diab, u_hr, u_sysbp, u_glucose, u_percoxyg, u_policy)

    return samples, actions2[:, 0], logp


# speedup vs baseline: 2.1623x; 2.1623x over previous
import functools

import jax
import jax.numpy as jnp
from jax.experimental import pallas as pl
from jax.experimental.pallas import tpu as pltpu

_EPS = 1e-10
_BIG = 2147483647


def _gmb(u):
    return -jnp.log(-jnp.log(u + _EPS) + _EPS)


def _body(dl_ref, pol_ref, t_hr, t_sbp, t_glu, t_po,
          ud_ref, u_hr, u_sbp, u_glu, u_po, up_ref,
          samples_ref, actions_ref, logp_ref,
          diab_s, *vs, V, C, N, B):
    groups = [tuple(vs[5 * k + j] for j in range(5)) for k in range(4)]
    i = pl.program_id(0)

    @pl.when(i == 0)
    def _init():
        dl = dl_ref[...]
        zd = dl + _gmb(ud_ref[...])
        s0 = (zd[:, 1:2] > zd[:, 0:1]).astype(jnp.int32)
        diab_s[...] = s0
        m2 = jnp.max(dl)
        lse2 = m2 + jnp.log(jnp.sum(jnp.exp(dl - m2)))
        samples_ref[:, 0:1] = s0
        logp_ref[:, 0:1] = jnp.where(s0 == 1, dl[0, 1], dl[0, 0]) - lse2

        pv = pol_ref[...]
        zp = pv + _gmb(up_ref[...])
        a = jnp.argmax(zp, axis=1).astype(jnp.int32)[:, None]
        actions_ref[...] = a
        mp = jnp.max(pv)
        lsep = mp + jnp.log(jnp.sum(jnp.exp(pv - mp)))
        ia8 = jax.lax.broadcasted_iota(jnp.int32, zp.shape, 1)
        tvp = jnp.sum(jnp.where(ia8 == a, pv, 0.0), axis=1, keepdims=True)
        logp_ref[:, 5:6] = tvp - lsep

        for (rmax, ridx, rtv, rm, rs) in groups:
            rmax[...] = jnp.full(rmax.shape, -jnp.inf, jnp.float32)
            ridx[...] = jnp.zeros(ridx.shape, jnp.int32)
            rtv[...] = jnp.zeros(rtv.shape, jnp.float32)
            rm[...] = jnp.full(rm.shape, -jnp.inf, jnp.float32)
            rs[...] = jnp.zeros(rs.shape, jnp.float32)

    diab = diab_s[...]
    base = i * C
    W = 128
    lane = jax.lax.broadcasted_iota(jnp.int32, (B, W), 1)
    col2 = jax.lax.broadcasted_iota(jnp.int32, (2, C), 1)
    valid2 = (base + col2) < V

    for (t_ref, u_ref), (rmax, ridx, rtv, rm, rs) in zip(
            [(t_hr, u_hr), (t_sbp, u_sbp), (t_glu, u_glu), (t_po, u_po)],
            groups):
        amax = rmax[...]
        aidx = ridx[...]
        atv = rtv[...]
        for k in range(C // W):
            off = k * W
            g = _gmb(u_ref[:, pl.ds(off, W)])
            t0 = t_ref[0:1, pl.ds(off, W)]
            t1 = t_ref[1:2, pl.ds(off, W)]
            tsel = jnp.where(diab == 1, t1, t0)
            col = (base + off) + lane
            z = jnp.where(col < V, tsel + g, -jnp.inf)
            upd = z > amax
            amax = jnp.where(upd, z, amax)
            aidx = jnp.where(upd, col, aidx)
            atv = jnp.where(upd, tsel, atv)
        rmax[...] = amax
        ridx[...] = aidx
        rtv[...] = atv
        t = t_ref[...]
        tmsk = jnp.where(valid2, t, -jnp.inf)
        tm = jnp.max(tmsk, axis=1, keepdims=True)
        m_old = rm[...]
        m_new = jnp.maximum(m_old, tm)
        se = jnp.sum(jnp.where(valid2, jnp.exp(t - m_new), 0.0),
                     axis=1, keepdims=True)
        rs[...] = rs[...] * jnp.exp(m_old - m_new) + se
        rm[...] = m_new

    @pl.when(i == N - 1)
    def _fin():
        diab_f = diab_s[...]
        for v, (rmax, ridx, rtv, rm, rs) in enumerate(groups):
            mv = rmax[...]
            iv = ridx[...]
            maxv = jnp.max(mv, axis=1, keepdims=True)
            at_max = mv == maxv
            bj = jnp.min(jnp.where(at_max, iv, _BIG),
                         axis=1, keepdims=True)
            sel_lane = at_max & (iv == bj)
            tv = jnp.sum(jnp.where(sel_lane, rtv[...], 0.0),
                         axis=1, keepdims=True)
            samples_ref[:, v + 1:v + 2] = bj
            lse = rm[...] + jnp.log(rs[...])
            lse_sel = jnp.where(diab_f == 1, lse[1, 0], lse[0, 0])
            logp_ref[:, v + 1:v + 2] = tv - lse_sel


def kernel(s0_diab_logits, s0_hr, s0_sysbp, s0_glucose, s0_percoxyg,
           policy_logits, u_diab, u_hr, u_sysbp, u_glucose, u_percoxyg,
           u_policy):
    B, V = u_hr.shape
    A = u_policy.shape[1]
    C = 4096
    N = pl.cdiv(V, C)
    dl = s0_diab_logits.reshape(1, 2)
    pol = policy_logits.reshape(1, A)

    const2 = lambda i: (0, 0)
    tspec = pl.BlockSpec((2, C), lambda i: (0, i))
    uspec = pl.BlockSpec((B, C), lambda i: (0, i))

    scratch = [pltpu.VMEM((B, 1), jnp.int32)]
    for _ in range(4):
        scratch += [pltpu.VMEM((B, 128), jnp.float32),
                    pltpu.VMEM((B, 128), jnp.int32),
                    pltpu.VMEM((B, 128), jnp.float32),
                    pltpu.VMEM((2, 1), jnp.float32),
                    pltpu.VMEM((2, 1), jnp.float32)]

    samples, actions2, logp = pl.pallas_call(
        functools.partial(_body, V=V, C=C, N=N, B=B),
        grid=(N,),
        in_specs=[
            pl.BlockSpec((1, 2), const2),
            pl.BlockSpec((1, A), const2),
            tspec, tspec, tspec, tspec,
            pl.BlockSpec((B, 2), const2),
            uspec, uspec, uspec, uspec,
            pl.BlockSpec((B, A), const2),
        ],
        out_specs=[
            pl.BlockSpec((B, 5), const2),
            pl.BlockSpec((B, 1), const2),
            pl.BlockSpec((B, 6), const2),
        ],
        out_shape=[
            jax.ShapeDtypeStruct((B, 5), jnp.int32),
            jax.ShapeDtypeStruct((B, 1), jnp.int32),
            jax.ShapeDtypeStruct((B, 6), jnp.float32),
        ],
        scratch_shapes=scratch,
        compiler_params=pltpu.CompilerParams(
            dimension_semantics=("arbitrary",)),
    )(dl, pol, s0_hr, s0_sysbp, s0_glucose, s0_percoxyg,
      u_diab, u_hr, u_sysbp, u_glucose, u_percoxyg, u_policy)

    return samples, actions2[:, 0], logp


# speedup vs baseline: 2.2122x; 1.0231x over previous
import functools

import jax
import jax.numpy as jnp
from jax.experimental import pallas as pl
from jax.experimental.pallas import tpu as pltpu

_EPS = 1e-10
_BIG = 2147483647


def _gmb(u):
    return -jnp.log(-jnp.log(u + _EPS) + _EPS)


def _body(dl_ref, pol_ref, t_hr, t_sbp, t_glu, t_po,
          ud_ref, u_hr, u_sbp, u_glu, u_po, up_ref,
          samples_ref, actions_ref, logp_ref,
          diab_s, *vs, V, C, N, B):
    groups = [tuple(vs[5 * k + j] for j in range(5)) for k in range(4)]
    i = pl.program_id(0)

    @pl.when(i == 0)
    def _init():
        dl = dl_ref[...]
        zd = dl + _gmb(ud_ref[...])
        s0 = (zd[:, 1:2] > zd[:, 0:1]).astype(jnp.int32)
        diab_s[...] = s0
        m2 = jnp.max(dl)
        lse2 = m2 + jnp.log(jnp.sum(jnp.exp(dl - m2)))
        samples_ref[:, 0:1] = s0
        logp_ref[:, 0:1] = jnp.where(s0 == 1, dl[0, 1], dl[0, 0]) - lse2

        pv = pol_ref[...]
        zp = pv + _gmb(up_ref[...])
        a = jnp.argmax(zp, axis=1).astype(jnp.int32)[:, None]
        actions_ref[...] = a
        mp = jnp.max(pv)
        lsep = mp + jnp.log(jnp.sum(jnp.exp(pv - mp)))
        ia8 = jax.lax.broadcasted_iota(jnp.int32, zp.shape, 1)
        tvp = jnp.sum(jnp.where(ia8 == a, pv, 0.0), axis=1, keepdims=True)
        logp_ref[:, 5:6] = tvp - lsep

        for (rmax, ridx, rtv, rm, rs) in groups:
            rmax[...] = jnp.full(rmax.shape, -jnp.inf, jnp.float32)
            ridx[...] = jnp.zeros(ridx.shape, jnp.int32)
            rtv[...] = jnp.zeros(rtv.shape, jnp.float32)
            rm[...] = jnp.full(rm.shape, -jnp.inf, jnp.float32)
            rs[...] = jnp.zeros(rs.shape, jnp.float32)

    dmask = diab_s[...] == 1
    base = i * C
    W = 128
    lane = jax.lax.broadcasted_iota(jnp.int32, (B, W), 1)
    rlane = jax.lax.broadcasted_iota(jnp.int32, (1, W), 1)
    col2 = jax.lax.broadcasted_iota(jnp.int32, (2, C), 1)
    valid2 = (base + col2) < V

    for (t_ref, u_ref), (rmax, ridx, rtv, rm, rs) in zip(
            [(t_hr, u_hr), (t_sbp, u_sbp), (t_glu, u_glu), (t_po, u_po)],
            groups):
        acc = [[rmax[...], ridx[...], rtv[...]],
               [jnp.full((B, W), -jnp.inf, jnp.float32),
                jnp.zeros((B, W), jnp.int32),
                jnp.zeros((B, W), jnp.float32)]]
        for k in range(C // W):
            off = k * W
            pen = jnp.where((base + off) + rlane < V, 0.0, -jnp.inf)
            g = _gmb(u_ref[:, pl.ds(off, W)])
            t0 = t_ref[0:1, pl.ds(off, W)] + pen
            t1 = t_ref[1:2, pl.ds(off, W)] + pen
            tsel = jnp.where(dmask, t1, t0)
            col = (base + off) + lane
            z = tsel + g
            amax, aidx, atv = acc[k % 2]
            upd = z > amax
            acc[k % 2][0] = jnp.where(upd, z, amax)
            acc[k % 2][1] = jnp.where(upd, col, aidx)
            acc[k % 2][2] = jnp.where(upd, tsel, atv)
        (ame, aie, ate), (amo, aio, ato) = acc
        take_o = (amo > ame) | ((amo == ame) & (aio < aie))
        rmax[...] = jnp.where(take_o, amo, ame)
        ridx[...] = jnp.where(take_o, aio, aie)
        rtv[...] = jnp.where(take_o, ato, ate)
        t = t_ref[...]
        tmsk = jnp.where(valid2, t, -jnp.inf)
        tm = jnp.max(tmsk, axis=1, keepdims=True)
        m_old = rm[...]
        m_new = jnp.maximum(m_old, tm)
        se = jnp.sum(jnp.where(valid2, jnp.exp(t - m_new), 0.0),
                     axis=1, keepdims=True)
        rs[...] = rs[...] * jnp.exp(m_old - m_new) + se
        rm[...] = m_new

    @pl.when(i == N - 1)
    def _fin():
        diab_f = diab_s[...]
        for v, (rmax, ridx, rtv, rm, rs) in enumerate(groups):
            mv = rmax[...]
            iv = ridx[...]
            maxv = jnp.max(mv, axis=1, keepdims=True)
            at_max = mv == maxv
            bj = jnp.min(jnp.where(at_max, iv, _BIG),
                         axis=1, keepdims=True)
            sel_lane = at_max & (iv == bj)
            tv = jnp.sum(jnp.where(sel_lane, rtv[...], 0.0),
                         axis=1, keepdims=True)
            samples_ref[:, v + 1:v + 2] = bj
            lse = rm[...] + jnp.log(rs[...])
            lse_sel = jnp.where(diab_f == 1, lse[1, 0], lse[0, 0])
            logp_ref[:, v + 1:v + 2] = tv - lse_sel


def kernel(s0_diab_logits, s0_hr, s0_sysbp, s0_glucose, s0_percoxyg,
           policy_logits, u_diab, u_hr, u_sysbp, u_glucose, u_percoxyg,
           u_policy):
    B, V = u_hr.shape
    A = u_policy.shape[1]
    C = 4096
    N = pl.cdiv(V, C)
    dl = s0_diab_logits.reshape(1, 2)
    pol = policy_logits.reshape(1, A)

    const2 = lambda i: (0, 0)
    tspec = pl.BlockSpec((2, C), lambda i: (0, i))
    uspec = pl.BlockSpec((B, C), lambda i: (0, i))

    scratch = [pltpu.VMEM((B, 1), jnp.int32)]
    for _ in range(4):
        scratch += [pltpu.VMEM((B, 128), jnp.float32),
                    pltpu.VMEM((B, 128), jnp.int32),
                    pltpu.VMEM((B, 128), jnp.float32),
                    pltpu.VMEM((2, 1), jnp.float32),
                    pltpu.VMEM((2, 1), jnp.float32)]

    samples, actions2, logp = pl.pallas_call(
        functools.partial(_body, V=V, C=C, N=N, B=B),
        grid=(N,),
        in_specs=[
            pl.BlockSpec((1, 2), const2),
            pl.BlockSpec((1, A), const2),
            tspec, tspec, tspec, tspec,
            pl.BlockSpec((B, 2), const2),
            uspec, uspec, uspec, uspec,
            pl.BlockSpec((B, A), const2),
        ],
        out_specs=[
            pl.BlockSpec((B, 5), const2),
            pl.BlockSpec((B, 1), const2),
            pl.BlockSpec((B, 6), const2),
        ],
        out_shape=[
            jax.ShapeDtypeStruct((B, 5), jnp.int32),
            jax.ShapeDtypeStruct((B, 1), jnp.int32),
            jax.ShapeDtypeStruct((B, 6), jnp.float32),
        ],
        scratch_shapes=scratch,
        compiler_params=pltpu.CompilerParams(
            dimension_semantics=("arbitrary",)),
    )(dl, pol, s0_hr, s0_sysbp, s0_glucose, s0_percoxyg,
      u_diab, u_hr, u_sysbp, u_glucose, u_percoxyg, u_policy)

    return samples, actions2[:, 0], logp
